# Initial kernel scaffold; baseline (speedup 1.0000x reference)
import jax
import jax.numpy as jnp
from jax.experimental import pallas as pl


def kernel(mu, current_edge_index, W1, b1, W2, b2, Ws1, bs1, Ws2, bs2, Wg1, bg1, Wg2, bg2):
    raise NotImplementedError("write your pallas kernel here")


# speedup vs baseline: 1414.5338x; 1414.5338x over previous
import functools

import jax
import jax.numpy as jnp
from jax import lax
from jax.experimental import pallas as pl
from jax.experimental.pallas import tpu as pltpu
from jax.experimental.pallas import tpu_sc as plsc

_B = 256
_D = 256
_H = 256
_E = 4096
_NEG = -1e30
_ROWS_PER_W = _B // 32


def _adj_body(edge_hbm, out_hbm, srcv, dstv, cbuf):
    wid = lax.axis_index("s") * 2 + lax.axis_index("c")
    lo = wid * _ROWS_PER_W
    pltpu.sync_copy(edge_hbm.at[0], srcv)
    pltpu.sync_copy(edge_hbm.at[1], dstv)
    zz = jnp.zeros((16,), jnp.float32)
    for i in range(_ROWS_PER_W * _B // 16):
        cbuf[pl.ds(i * 16, 16)] = zz
    ones = jnp.ones((16,), jnp.float32)

    def step(e, carry):
        s = srcv[pl.ds(e * 16, 16)]
        d = dstv[pl.ds(e * 16, 16)]
        m = (d >= lo) & (d < lo + _ROWS_PER_W)
        flat = jnp.where(m, (d - lo) * _B + s, 0)
        plsc.addupdate_scatter(cbuf, [flat], ones, mask=m)
        return carry

    lax.fori_loop(0, _E // 16, step, 0)
    for r in range(_ROWS_PER_W):
        pltpu.sync_copy(cbuf.at[pl.ds(r * _B, _B)], out_hbm.at[lo + r])


def _adj_call(edge_index):
    mesh = plsc.VectorSubcoreMesh(core_axis_name="c", subcore_axis_name="s")
    k = functools.partial(
        pl.kernel,
        mesh=mesh,
        compiler_params=pltpu.CompilerParams(
            use_tc_tiling_on_sc=False, needs_layout_passes=False),
        out_type=jax.ShapeDtypeStruct((_B, _B), jnp.float32),
        scratch_types=[
            pltpu.VMEM((_E,), jnp.int32),
            pltpu.VMEM((_E,), jnp.int32),
            pltpu.VMEM((_ROWS_PER_W * _B,), jnp.float32),
        ],
    )(_adj_body)
    return k(edge_index)


def _tc_body(mu, C, W1, b1, W2, b2, Ws1f, bs1, Ws2, bs2,
             Wg1a, Wg1b, bg1, Wg2, bg2, meta_out, gamma_out):
    f32 = jnp.float32
    dot = functools.partial(jnp.dot, preferred_element_type=f32,
                            precision=lax.Precision.HIGHEST)
    bdot = functools.partial(jnp.dot, preferred_element_type=f32)
    II = lax.broadcasted_iota(jnp.int32, (_B, _B), 0).astype(f32)
    JJ = lax.broadcasted_iota(jnp.int32, (_B, _B), 1).astype(f32)
    EY = (II == JJ).astype(f32)

    def tp(M):
        return lax.dot_general(M[...], EY, (((0,), (0,)), ((), ())),
                               precision=lax.Precision.HIGHEST,
                               preferred_element_type=f32)

    bf = jnp.bfloat16
    muv = mu[...]
    Cv = C[...]
    deg = jnp.sum(Cv, axis=1, keepdims=True) + 1.0
    dinv = 1.0 / jnp.sqrt(deg)
    Nrm = (Cv + EY) * dinv * tp(dinv)
    xw1 = bdot(muv.astype(bf), W1[...].astype(bf))
    h1 = jax.nn.relu(dot(Nrm, xw1) + b1[...])
    xw2 = bdot(h1.astype(bf), W2[...].astype(bf))
    h = jax.nn.relu(dot(Nrm, xw2) + b2[...])

    hb = h.astype(bf)
    Ws1bf = Ws1f[...].astype(bf)
    Ws2bf = Ws2[...].astype(bf)
    bs1v = bs1[...]
    cols = []
    for c in range(_B // 8):
        feats = jnp.concatenate(
            [jnp.concatenate(
                [jnp.broadcast_to(hb[8 * c + p:8 * c + p + 1, :], (_B, _H)),
                 hb], axis=1)
             for p in range(8)], axis=0)
        hid = jax.nn.relu(bdot(feats, Ws1bf) + bs1v)
        hidb = hid.astype(bf)
        for p in range(8):
            cols.append(bdot(hidb[_B * p:_B * (p + 1), :], Ws2bf))
    St = jnp.concatenate(cols, axis=1)
    S = tp(St) + bs2[...]
    upper = jnp.where(II < JJ, S, 0.0)
    Ssym = upper + tp(upper)

    cidx = lax.broadcasted_iota(jnp.int32, (_B, 1), 0).astype(f32)
    comp = cidx
    comp_row = tp(comp)
    ei_list, ej_list, es_list = [], [], []
    for _ in range(8):
        mask_same = comp == comp_row
        W = jnp.where(mask_same, _NEG, Ssym)
        bestval = jnp.max(W, axis=1, keepdims=True)
        bestu = jnp.min(jnp.where(W == bestval, JJ, 300.0),
                        axis=1, keepdims=True)
        bestval_row = tp(bestval)
        condC = comp_row == II
        G = jnp.where(condC, bestval_row, _NEG)
        compmax = jnp.max(G, axis=1, keepdims=True)
        active = compmax > -1e29
        wv = jnp.min(jnp.where(condC & (bestval_row == compmax), JJ, 300.0),
                     axis=1, keepdims=True)
        wv = jnp.where(active, wv, 0.0)
        bu = dot((JJ == wv).astype(f32), bestu)
        bu = jnp.where(active, bu, 0.0)
        pc = dot((JJ == bu).astype(f32), comp)
        link = jnp.where(active, pc, cidx)
        ll = dot((JJ == link).astype(f32), link)
        two_cycle = ll == cidx
        emit = active & jnp.logical_not(two_cycle & (link < cidx))
        ei_list.append(jnp.where(emit, jnp.minimum(wv, bu), 0.0))
        ej_list.append(jnp.where(emit, jnp.maximum(wv, bu), 0.0))
        es_list.append(jnp.where(emit, compmax, _NEG))
        l2 = jnp.where(two_cycle, jnp.minimum(cidx, link), link)
        for _ in range(8):
            l2 = dot((JJ == l2).astype(f32), l2)
        comp = dot((JJ == comp).astype(f32), l2)
        comp_row = tp(comp)

    es_all = jnp.concatenate(es_list, axis=0)
    kidx = lax.broadcasted_iota(jnp.int32, (8 * _B, 1), 0).astype(f32)
    lrow = lax.broadcasted_iota(jnp.int32, (1, _B), 1).astype(f32)
    rank = jnp.zeros((8 * _B, 1), f32)
    for r in range(8):
        es_row = tp(es_list[r])
        lidx = lrow + (r * _B)
        gt = (es_row > es_all) | ((es_row == es_all) & (lidx < kidx))
        rank = rank + jnp.sum(gt.astype(f32), axis=1, keepdims=True)
    i_idx = jnp.zeros((_B, 1), f32)
    j_idx = jnp.zeros((_B, 1), f32)
    for r in range(8):
        rrow = tp(rank[r * _B:(r + 1) * _B, :])
        OH = (rrow == II).astype(f32)
        i_idx = i_idx + dot(OH, ei_list[r])
        j_idx = j_idx + dot(OH, ej_list[r])

    Hi = dot((JJ == i_idx).astype(f32), h)
    Hj = dot((JJ == j_idx).astype(f32), h)
    g1 = jax.nn.relu(dot(Hi, Wg1a[...]) + dot(Hj, Wg1b[...]) + bg1[...])
    gamma_out[...] = jnp.tanh(dot(g1, Wg2[...]) + bg2[...])

    meta = jnp.concatenate(
        [tp(i_idx), tp(j_idx), jnp.zeros((6, _B), f32)], axis=0)
    meta_out[...] = meta.astype(jnp.int32)


def _tc_call(*args, interpret=False):
    return pl.pallas_call(
        _tc_body,
        out_shape=(
            jax.ShapeDtypeStruct((8, _B), jnp.int32),
            jax.ShapeDtypeStruct((_B, _D), jnp.float32),
        ),
        interpret=interpret,
    )(*args)


def kernel(mu, current_edge_index, W1, b1, W2, b2, Ws1, bs1, Ws2, bs2,
           Wg1, bg1, Wg2, bg2):
    edge = current_edge_index.astype(jnp.int32)
    C = _adj_call(edge)
    meta, gamma_full = _tc_call(
        mu, C, W1, b1.reshape(1, _H), W2, b2.reshape(1, _H),
        Ws1, bs1.reshape(1, _H), Ws2, bs2.reshape(1, 1),
        Wg1[:_H], Wg1[_H:], bg1.reshape(1, _H), Wg2,
        bg2.reshape(1, _D))
    i_idx = meta[0, :_B - 1]
    j_idx = meta[1, :_B - 1]
    src_new = jnp.stack([i_idx, j_idx], axis=1).reshape(-1)
    dst_new = jnp.stack([j_idx, i_idx], axis=1).reshape(-1)
    new_edge_index = jnp.stack([src_new, dst_new], axis=0)
    return (new_edge_index, gamma_full[:_B - 1])
